# baseline (speedup 1.0000x reference)
import functools

import jax
import jax.numpy as jnp
from jax import lax
from jax.experimental import pallas as pl
from jax.experimental.pallas import tpu as pltpu
from jax.experimental.pallas import tpu_sc as plsc

_M, _C, _N, _K = 2, 128, 10000, 32
_NW = 32
_ROWS_W = (_M * _N) // _NW
_FB = 4
_NCHUNK = _ROWS_W // _FB
_LANES = 16


_CB = 128
_NFULL = (_M * _N) // _CB
_TAIL = _M * _N - _NFULL * _CB
_NCHUNKS = _NFULL + 1


_NSLOT = _NFULL // _NW + 1


def _gather_sum_body(fea_hbm, ringT_hbm, out_hbm, idx_v, acc0, acc1, idx_t,
                     sem_idx, sem_init, sem_add0, sem_add1):
    wid = lax.axis_index("s") * 2 + lax.axis_index("c")
    accs = [acc0, acc1]

    def staging(s):
        chunk = jnp.minimum(wid + _NW * s, _NFULL - 1)
        buf = s % 2
        idx_cp = pltpu.make_async_copy(
            ringT_hbm.at[:, pl.ds(chunk * _CB, _CB)], idx_v.at[buf], sem_idx)
        init_cp = pltpu.make_async_copy(
            fea_hbm.at[pl.ds(chunk * _CB, _CB)], accs[buf], sem_init)
        return idx_cp, init_cp

    descs = [staging(s) for s in range(_NSLOT)]
    descs[0][0].start()
    descs[0][1].start()
    adds = [None] * _NSLOT
    v_last = wid + _NW * (_NSLOT - 1) < _NFULL

    def fire_adds(s):
        buf = s % 2
        descs[s][0].wait()
        descs[s][1].wait()
        adds[s] = [
            pltpu.async_copy(fea_hbm.at[idx_v.at[buf, j]], accs[buf],
                             [sem_add0, sem_add1][buf], add=True)
            for j in range(_K)
        ]

    def retire(s):
        for cp in adds[s]:
            cp.wait()
        pltpu.sync_copy(accs[s % 2],
                        out_hbm.at[pl.ds((wid + _NW * s) * _CB, _CB)])

    for s in range(_NSLOT - 1):
        fire_adds(s)
        if s >= 1:
            retire(s - 1)
        if s + 1 == _NSLOT - 1:
            @pl.when(v_last)
            def _stage_last():
                descs[s + 1][0].start()
                descs[s + 1][1].start()
        else:
            descs[s + 1][0].start()
            descs[s + 1][1].start()

    sl = _NSLOT - 1

    @pl.when(v_last)
    def _last_slot():
        fire_adds(sl)
        retire(sl - 1)
        retire(sl)

    @pl.when(jnp.logical_not(v_last))
    def _drain():
        retire(sl - 1)

    @pl.when(wid == _NFULL % _NW)
    def _tail():
        r0 = _NFULL * _CB
        pltpu.sync_copy(ringT_hbm.at[:, pl.ds(r0, _TAIL)], idx_t)
        pltpu.sync_copy(fea_hbm.at[pl.ds(r0, _TAIL)], acc0.at[pl.ds(0, _TAIL)])
        adds = [
            pltpu.async_copy(fea_hbm.at[idx_t.at[j]],
                             acc0.at[pl.ds(0, _TAIL)], sem_add0, add=True)
            for j in range(_K)
        ]
        for cp in adds:
            cp.wait()
        pltpu.sync_copy(acc0.at[pl.ds(0, _TAIL)], out_hbm.at[pl.ds(r0, _TAIL)])


@functools.cache
def _gather_sum():
    return pl.kernel(
        _gather_sum_body,
        mesh=plsc.VectorSubcoreMesh(core_axis_name="c", subcore_axis_name="s"),
        compiler_params=pltpu.CompilerParams(use_tc_tiling_on_sc=False),
        out_type=jax.ShapeDtypeStruct((_M * _N, _C), jnp.float32),
        scratch_types=[
            pltpu.VMEM((2, _K, _CB), jnp.int32),
            pltpu.VMEM((_CB, _C), jnp.float32),
            pltpu.VMEM((_CB, _C), jnp.float32),
            pltpu.VMEM((_K, _TAIL), jnp.int32),
            pltpu.SemaphoreType.DMA,
            pltpu.SemaphoreType.DMA,
            pltpu.SemaphoreType.DMA,
            pltpu.SemaphoreType.DMA,
        ],
    )


_NMOM = 10


def _post_body(sm_ref, sc_ref, wt_ref, b_ref, g_ref, beta_ref, o_ref,
               ssum_ref, scov_ref, wft_ref, bf_ref):
    i = pl.program_id(0)

    @pl.when(i == 0)
    def _init():
        ssum_ref[...] = jnp.zeros_like(ssum_ref)
        scov_ref[...] = jnp.zeros_like(scov_ref)

    @pl.when(i < _NMOM)
    def _acc():
        s = sm_ref[...]
        ssum_ref[...] += jnp.sum(s, axis=0, keepdims=True)
        scov_ref[...] += lax.dot_general(
            s, s, (((0,), (0,)), ((), ())), preferred_element_type=jnp.float32)

    @pl.when(i == _NMOM - 1)
    def _fold():
        cnt = float(_M * _N)
        wt = wt_ref[...]
        mean_s = ssum_ref[...] / cnt
        outer = lax.dot_general(mean_s, mean_s, (((0,), (0,)), ((), ())),
                                preferred_element_type=jnp.float32)
        cov = scov_ref[...] / cnt - outer
        mu = lax.dot_general(mean_s, wt, (((1,), (0,)), ((), ())),
                             preferred_element_type=jnp.float32) + b_ref[...]
        gt = lax.dot_general(cov, wt, (((1,), (0,)), ((), ())),
                             preferred_element_type=jnp.float32)
        ones = jnp.ones((1, _C), jnp.float32)
        var = lax.dot_general(ones, gt * wt, (((1,), (0,)), ((), ())),
                              preferred_element_type=jnp.float32)
        scale = g_ref[...] * lax.rsqrt(var + 1e-5)
        wft_ref[...] = wt * scale
        bf_ref[...] = (b_ref[...] - mu) * scale + beta_ref[...]

    @pl.when(i >= _NMOM)
    def _conv():
        s = sc_ref[0]
        yT = lax.dot_general(
            wft_ref[...], s, (((0,), (1,)), ((), ())),
            preferred_element_type=jnp.float32)
        ones_n = jnp.ones((1, _N), jnp.float32)
        bias = lax.dot_general(bf_ref[...], ones_n, (((0,), (0,)), ((), ())),
                               preferred_element_type=jnp.float32)
        o_ref[0] = jnp.maximum(yT + bias, 0.0)


def _post(summed, conv_w, conv_b, bn_gamma, bn_beta):
    blk = (_M * _N) // _NMOM
    cc = pl.BlockSpec((_C, _C), lambda i: (0, 0))
    r1 = pl.BlockSpec((1, _C), lambda i: (0, 0))
    return pl.pallas_call(
        _post_body,
        grid=(_NMOM + _M,),
        in_specs=[
            pl.BlockSpec((blk, _C), lambda i: (jnp.minimum(i, _NMOM - 1), 0)),
            pl.BlockSpec((1, _N, _C),
                         lambda i: (jnp.maximum(i - _NMOM, 0), 0, 0)),
            cc, r1, r1, r1,
        ],
        out_specs=pl.BlockSpec((1, _C, _N),
                               lambda i: (jnp.maximum(i - _NMOM, 0), 0, 0)),
        out_shape=jax.ShapeDtypeStruct((_M, _C, _N), jnp.float32),
        scratch_shapes=[
            pltpu.VMEM((1, _C), jnp.float32),
            pltpu.VMEM((_C, _C), jnp.float32),
            pltpu.VMEM((_C, _C), jnp.float32),
            pltpu.VMEM((1, _C), jnp.float32),
        ],
    )(summed, summed.reshape(_M, _N, _C), conv_w.T,
      conv_b[None, :], bn_gamma[None, :], bn_beta[None, :])


def kernel(fea, ring_n, pool_idx, pos_embed, conv_w, conv_b, bn_gamma, bn_beta):
    fea_t = jnp.transpose(fea, (0, 2, 1)).reshape(_M * _N, _C)
    ring_t = (ring_n.astype(jnp.int32) + (jnp.arange(_M, dtype=jnp.int32) * _N)[:, None, None])
    ring_t = jnp.transpose(ring_t, (2, 0, 1)).reshape(_K, _M * _N)

    summed = _gather_sum()(fea_t, ring_t)
    return _post(summed, conv_w, conv_b, bn_gamma, bn_beta)


# speedup vs baseline: 1.0200x; 1.0200x over previous
import functools

import jax
import jax.numpy as jnp
from jax import lax
from jax.experimental import pallas as pl
from jax.experimental.pallas import tpu as pltpu
from jax.experimental.pallas import tpu_sc as plsc

_M, _C, _N, _K = 2, 128, 10000, 32
_NW = 32
_ROWS_W = (_M * _N) // _NW
_FB = 4
_NCHUNK = _ROWS_W // _FB
_LANES = 16


_CB = 128
_NFULL = (_M * _N) // _CB
_TAIL = _M * _N - _NFULL * _CB
_NCHUNKS = _NFULL + 1


_NSLOT = _NFULL // _NW + 1


def _gather_sum_body(fea_hbm, ringT_hbm, out_hbm, idx_v, acc0, acc1, idx_t,
                     sem_idx, sem_init, sem_add0, sem_add1):
    wid = lax.axis_index("s") * 2 + lax.axis_index("c")
    accs = [acc0, acc1]

    def staging(s):
        chunk = jnp.minimum(wid + _NW * s, _NFULL - 1)
        buf = s % 2
        idx_cp = pltpu.make_async_copy(
            ringT_hbm.at[:, pl.ds(chunk * _CB, _CB)], idx_v.at[buf], sem_idx)
        init_cp = pltpu.make_async_copy(
            fea_hbm.at[pl.ds(chunk * _CB, _CB)], accs[buf], sem_init)
        return idx_cp, init_cp

    descs = [staging(s) for s in range(_NSLOT)]
    descs[0][0].start()
    descs[0][1].start()
    out_cps = [None] * _NSLOT

    def run_slot(s, last):
        chunk = wid + _NW * s
        buf = s % 2
        descs[s][0].wait()
        descs[s][1].wait()
        adds = [
            pltpu.async_copy(fea_hbm.at[idx_v.at[buf, j]], accs[buf],
                             sem_add0, add=True)
            for j in range(_K)
        ]
        if s >= 1:
            out_cps[s - 1].wait()
        if s + 1 < _NSLOT:
            if s + 1 == _NSLOT - 1:
                @pl.when(wid + _NW * (s + 1) < _NFULL)
                def _stage_last():
                    descs[s + 1][0].start()
                    descs[s + 1][1].start()
            else:
                descs[s + 1][0].start()
                descs[s + 1][1].start()
        for cp in adds:
            cp.wait()
        dst = out_hbm.at[pl.ds(chunk * _CB, _CB)]
        if last:
            pltpu.sync_copy(accs[buf], dst)
        else:
            out_cps[s] = pltpu.make_async_copy(accs[buf], dst, sem_add1)
            out_cps[s].start()

    for s in range(_NSLOT - 1):
        run_slot(s, last=False)

    v_last = wid + _NW * (_NSLOT - 1) < _NFULL

    @pl.when(v_last)
    def _last_slot():
        run_slot(_NSLOT - 1, last=True)

    @pl.when(jnp.logical_not(v_last))
    def _drain():
        out_cps[_NSLOT - 2].wait()

    @pl.when(wid == _NFULL % _NW)
    def _tail():
        r0 = _NFULL * _CB
        pltpu.sync_copy(ringT_hbm.at[:, pl.ds(r0, _TAIL)], idx_t)
        pltpu.sync_copy(fea_hbm.at[pl.ds(r0, _TAIL)], acc0.at[pl.ds(0, _TAIL)])
        adds = [
            pltpu.async_copy(fea_hbm.at[idx_t.at[j]],
                             acc0.at[pl.ds(0, _TAIL)], sem_add0, add=True)
            for j in range(_K)
        ]
        for cp in adds:
            cp.wait()
        pltpu.sync_copy(acc0.at[pl.ds(0, _TAIL)], out_hbm.at[pl.ds(r0, _TAIL)])


@functools.cache
def _gather_sum():
    return pl.kernel(
        _gather_sum_body,
        mesh=plsc.VectorSubcoreMesh(core_axis_name="c", subcore_axis_name="s"),
        compiler_params=pltpu.CompilerParams(use_tc_tiling_on_sc=False),
        out_type=jax.ShapeDtypeStruct((_M * _N, _C), jnp.float32),
        scratch_types=[
            pltpu.VMEM((2, _K, _CB), jnp.int32),
            pltpu.VMEM((_CB, _C), jnp.float32),
            pltpu.VMEM((_CB, _C), jnp.float32),
            pltpu.VMEM((_K, _TAIL), jnp.int32),
            pltpu.SemaphoreType.DMA,
            pltpu.SemaphoreType.DMA,
            pltpu.SemaphoreType.DMA,
            pltpu.SemaphoreType.DMA,
        ],
    )


_NMOM = 10


def _post_body(sm_ref, sc_ref, wt_ref, b_ref, g_ref, beta_ref, o_ref,
               ssum_ref, scov_ref, wft_ref, bf_ref):
    i = pl.program_id(0)

    @pl.when(i == 0)
    def _init():
        ssum_ref[...] = jnp.zeros_like(ssum_ref)
        scov_ref[...] = jnp.zeros_like(scov_ref)

    @pl.when(i < _NMOM)
    def _acc():
        s = sm_ref[...]
        ssum_ref[...] += jnp.sum(s, axis=0, keepdims=True)
        scov_ref[...] += lax.dot_general(
            s, s, (((0,), (0,)), ((), ())), preferred_element_type=jnp.float32)

    @pl.when(i == _NMOM - 1)
    def _fold():
        cnt = float(_M * _N)
        wt = wt_ref[...]
        mean_s = ssum_ref[...] / cnt
        outer = lax.dot_general(mean_s, mean_s, (((0,), (0,)), ((), ())),
                                preferred_element_type=jnp.float32)
        cov = scov_ref[...] / cnt - outer
        mu = lax.dot_general(mean_s, wt, (((1,), (0,)), ((), ())),
                             preferred_element_type=jnp.float32) + b_ref[...]
        gt = lax.dot_general(cov, wt, (((1,), (0,)), ((), ())),
                             preferred_element_type=jnp.float32)
        ones = jnp.ones((1, _C), jnp.float32)
        var = lax.dot_general(ones, gt * wt, (((1,), (0,)), ((), ())),
                              preferred_element_type=jnp.float32)
        scale = g_ref[...] * lax.rsqrt(var + 1e-5)
        wft_ref[...] = wt * scale
        bf_ref[...] = (b_ref[...] - mu) * scale + beta_ref[...]

    @pl.when(i >= _NMOM)
    def _conv():
        s = sc_ref[0]
        yT = lax.dot_general(
            wft_ref[...], s, (((0,), (1,)), ((), ())),
            preferred_element_type=jnp.float32)
        ones_n = jnp.ones((1, _N), jnp.float32)
        bias = lax.dot_general(bf_ref[...], ones_n, (((0,), (0,)), ((), ())),
                               preferred_element_type=jnp.float32)
        o_ref[0] = jnp.maximum(yT + bias, 0.0)


def _post(summed, conv_w, conv_b, bn_gamma, bn_beta):
    blk = (_M * _N) // _NMOM
    cc = pl.BlockSpec((_C, _C), lambda i: (0, 0))
    r1 = pl.BlockSpec((1, _C), lambda i: (0, 0))
    return pl.pallas_call(
        _post_body,
        grid=(_NMOM + _M,),
        in_specs=[
            pl.BlockSpec((blk, _C), lambda i: (jnp.minimum(i, _NMOM - 1), 0)),
            pl.BlockSpec((1, _N, _C),
                         lambda i: (jnp.maximum(i - _NMOM, 0), 0, 0)),
            cc, r1, r1, r1,
        ],
        out_specs=pl.BlockSpec((1, _C, _N),
                               lambda i: (jnp.maximum(i - _NMOM, 0), 0, 0)),
        out_shape=jax.ShapeDtypeStruct((_M, _C, _N), jnp.float32),
        scratch_shapes=[
            pltpu.VMEM((1, _C), jnp.float32),
            pltpu.VMEM((_C, _C), jnp.float32),
            pltpu.VMEM((_C, _C), jnp.float32),
            pltpu.VMEM((1, _C), jnp.float32),
        ],
    )(summed, summed.reshape(_M, _N, _C), conv_w.T,
      conv_b[None, :], bn_gamma[None, :], bn_beta[None, :])


def kernel(fea, ring_n, pool_idx, pos_embed, conv_w, conv_b, bn_gamma, bn_beta):
    fea_t = jnp.transpose(fea, (0, 2, 1)).reshape(_M * _N, _C)
    ring_t = (ring_n.astype(jnp.int32) + (jnp.arange(_M, dtype=jnp.int32) * _N)[:, None, None])
    ring_t = jnp.transpose(ring_t, (2, 0, 1)).reshape(_K, _M * _N)

    summed = _gather_sum()(fea_t, ring_t)
    return _post(summed, conv_w, conv_b, bn_gamma, bn_beta)


# speedup vs baseline: 1.0326x; 1.0124x over previous
import functools

import jax
import jax.numpy as jnp
from jax import lax
from jax.experimental import pallas as pl
from jax.experimental.pallas import tpu as pltpu
from jax.experimental.pallas import tpu_sc as plsc

_M, _C, _N, _K = 2, 128, 10000, 32
_NW = 32
_ROWS_W = (_M * _N) // _NW
_FB = 4
_NCHUNK = _ROWS_W // _FB
_LANES = 16


_CB = 128
_NFULL = (_M * _N) // _CB
_TAIL = _M * _N - _NFULL * _CB
_NCHUNKS = _NFULL + 1


_NSLOT = _NFULL // _NW + 1


def _gather_sum_body(fea_hbm, ringT_hbm, out_hbm, idx_v, acc0, acc1, idx_t,
                     sem_idx, sem_init, sem_add0, sem_add1):
    wid = lax.axis_index("s") * 2 + lax.axis_index("c")
    accs = [acc0, acc1]

    def staging(s):
        chunk = jnp.minimum(wid + _NW * s, _NFULL - 1)
        buf = s % 2
        idx_cp = pltpu.make_async_copy(
            ringT_hbm.at[:, pl.ds(chunk * _CB, _CB)], idx_v.at[buf], sem_idx)
        init_cp = pltpu.make_async_copy(
            fea_hbm.at[pl.ds(chunk * _CB, _CB)], accs[buf], sem_init)
        return idx_cp, init_cp

    descs = [staging(s) for s in range(_NSLOT)]
    descs[0][0].start()
    descs[0][1].start()
    out_cps = [None] * _NSLOT

    def run_slot(s, last):
        chunk = wid + _NW * s
        buf = s % 2
        descs[s][0].wait()
        descs[s][1].wait()
        adds = [
            pltpu.async_copy(fea_hbm.at[idx_v.at[buf, j]], accs[buf],
                             sem_add0, add=True)
            for j in range(_K)
        ]
        if s >= 1:
            out_cps[s - 1].wait()
        if s + 1 < _NSLOT:
            if s + 1 == _NSLOT - 1:
                @pl.when(wid + _NW * (s + 1) < _NFULL)
                def _stage_last():
                    descs[s + 1][0].start()
                    descs[s + 1][1].start()
            else:
                descs[s + 1][0].start()
                descs[s + 1][1].start()
        for cp in adds:
            cp.wait()
        dst = out_hbm.at[pl.ds(chunk * _CB, _CB)]
        if last:
            pltpu.sync_copy(accs[buf], dst)
        else:
            out_cps[s] = pltpu.make_async_copy(accs[buf], dst, sem_add1)
            out_cps[s].start()

    for s in range(_NSLOT - 1):
        run_slot(s, last=False)

    v_last = wid + _NW * (_NSLOT - 1) < _NFULL

    @pl.when(v_last)
    def _last_slot():
        run_slot(_NSLOT - 1, last=True)

    @pl.when(jnp.logical_not(v_last))
    def _drain():
        out_cps[_NSLOT - 2].wait()

    @pl.when(wid == _NFULL % _NW)
    def _tail():
        r0 = _NFULL * _CB
        pltpu.sync_copy(ringT_hbm.at[:, pl.ds(r0, _TAIL)], idx_t)
        pltpu.sync_copy(fea_hbm.at[pl.ds(r0, _TAIL)], acc0.at[pl.ds(0, _TAIL)])
        adds = [
            pltpu.async_copy(fea_hbm.at[idx_t.at[j]],
                             acc0.at[pl.ds(0, _TAIL)], sem_add0, add=True)
            for j in range(_K)
        ]
        for cp in adds:
            cp.wait()
        pltpu.sync_copy(acc0.at[pl.ds(0, _TAIL)], out_hbm.at[pl.ds(r0, _TAIL)])


@functools.cache
def _gather_sum():
    return pl.kernel(
        _gather_sum_body,
        mesh=plsc.VectorSubcoreMesh(core_axis_name="c", subcore_axis_name="s"),
        compiler_params=pltpu.CompilerParams(use_tc_tiling_on_sc=False),
        out_type=jax.ShapeDtypeStruct((_M * _N, _C), jnp.float32),
        scratch_types=[
            pltpu.VMEM((2, _K, _CB), jnp.int32),
            pltpu.VMEM((_CB, _C), jnp.float32),
            pltpu.VMEM((_CB, _C), jnp.float32),
            pltpu.VMEM((_K, _TAIL), jnp.int32),
            pltpu.SemaphoreType.DMA,
            pltpu.SemaphoreType.DMA,
            pltpu.SemaphoreType.DMA,
            pltpu.SemaphoreType.DMA,
        ],
    )


_NMOM = 5


def _post_body(sm_ref, sc_ref, wt_ref, b_ref, g_ref, beta_ref, o_ref,
               ssum_ref, scov_ref, wft_ref, bf_ref):
    i = pl.program_id(0)

    @pl.when(i == 0)
    def _init():
        ssum_ref[...] = jnp.zeros_like(ssum_ref)
        scov_ref[...] = jnp.zeros_like(scov_ref)

    @pl.when(i < _NMOM)
    def _acc():
        s = sm_ref[...]
        ssum_ref[...] += jnp.sum(s, axis=0, keepdims=True)
        scov_ref[...] += lax.dot_general(
            s, s, (((0,), (0,)), ((), ())), preferred_element_type=jnp.float32)

    @pl.when(i == _NMOM - 1)
    def _fold():
        cnt = float(_M * _N)
        wt = wt_ref[...]
        mean_s = ssum_ref[...] / cnt
        outer = lax.dot_general(mean_s, mean_s, (((0,), (0,)), ((), ())),
                                preferred_element_type=jnp.float32)
        cov = scov_ref[...] / cnt - outer
        mu = lax.dot_general(mean_s, wt, (((1,), (0,)), ((), ())),
                             preferred_element_type=jnp.float32) + b_ref[...]
        gt = lax.dot_general(cov, wt, (((1,), (0,)), ((), ())),
                             preferred_element_type=jnp.float32)
        ones = jnp.ones((1, _C), jnp.float32)
        var = lax.dot_general(ones, gt * wt, (((1,), (0,)), ((), ())),
                              preferred_element_type=jnp.float32)
        scale = g_ref[...] * lax.rsqrt(var + 1e-5)
        wft_ref[...] = wt * scale
        bf_ref[...] = (b_ref[...] - mu) * scale + beta_ref[...]

    @pl.when(i >= _NMOM)
    def _conv():
        s = sc_ref[0]
        yT = lax.dot_general(
            wft_ref[...], s, (((0,), (1,)), ((), ())),
            preferred_element_type=jnp.float32)
        ones_n = jnp.ones((1, _N), jnp.float32)
        bias = lax.dot_general(bf_ref[...], ones_n, (((0,), (0,)), ((), ())),
                               preferred_element_type=jnp.float32)
        o_ref[0] = jnp.maximum(yT + bias, 0.0)


def _post(summed, conv_w, conv_b, bn_gamma, bn_beta):
    blk = (_M * _N) // _NMOM
    cc = pl.BlockSpec((_C, _C), lambda i: (0, 0))
    r1 = pl.BlockSpec((1, _C), lambda i: (0, 0))
    return pl.pallas_call(
        _post_body,
        grid=(_NMOM + _M,),
        in_specs=[
            pl.BlockSpec((blk, _C), lambda i: (jnp.minimum(i, _NMOM - 1), 0)),
            pl.BlockSpec((1, _N, _C),
                         lambda i: (jnp.maximum(i - _NMOM, 0), 0, 0)),
            cc, r1, r1, r1,
        ],
        out_specs=pl.BlockSpec((1, _C, _N),
                               lambda i: (jnp.maximum(i - _NMOM, 0), 0, 0)),
        out_shape=jax.ShapeDtypeStruct((_M, _C, _N), jnp.float32),
        scratch_shapes=[
            pltpu.VMEM((1, _C), jnp.float32),
            pltpu.VMEM((_C, _C), jnp.float32),
            pltpu.VMEM((_C, _C), jnp.float32),
            pltpu.VMEM((1, _C), jnp.float32),
        ],
    )(summed, summed.reshape(_M, _N, _C), conv_w.T,
      conv_b[None, :], bn_gamma[None, :], bn_beta[None, :])


def kernel(fea, ring_n, pool_idx, pos_embed, conv_w, conv_b, bn_gamma, bn_beta):
    fea_t = jnp.transpose(fea, (0, 2, 1)).reshape(_M * _N, _C)
    ring_t = (ring_n.astype(jnp.int32) + (jnp.arange(_M, dtype=jnp.int32) * _N)[:, None, None])
    ring_t = jnp.transpose(ring_t, (2, 0, 1)).reshape(_K, _M * _N)

    summed = _gather_sum()(fea_t, ring_t)
    return _post(summed, conv_w, conv_b, bn_gamma, bn_beta)


# speedup vs baseline: 1.0407x; 1.0078x over previous
import functools

import jax
import jax.numpy as jnp
from jax import lax
from jax.experimental import pallas as pl
from jax.experimental.pallas import tpu as pltpu
from jax.experimental.pallas import tpu_sc as plsc

_M, _C, _N, _K = 2, 128, 10000, 32
_NW = 32
_ROWS_W = (_M * _N) // _NW
_FB = 4
_NCHUNK = _ROWS_W // _FB
_LANES = 16


_CB = 128
_NFULL = (_M * _N) // _CB
_TAIL = _M * _N - _NFULL * _CB
_NCHUNKS = _NFULL + 1


_NSLOT = _NFULL // _NW + 1


def _gather_sum_body(fea_hbm, ringT_hbm, out_hbm, idx_v, acc0, acc1, idx_t,
                     sem_idx, sem_init, sem_add0, sem_add1):
    wid = lax.axis_index("s") * 2 + lax.axis_index("c")
    accs = [acc0, acc1]

    def staging(s):
        chunk = jnp.minimum(wid + _NW * s, _NFULL - 1)
        buf = s % 2
        idx_cp = pltpu.make_async_copy(
            ringT_hbm.at[:, pl.ds(chunk * _CB, _CB)], idx_v.at[buf], sem_idx)
        init_cp = pltpu.make_async_copy(
            fea_hbm.at[pl.ds(chunk * _CB, _CB)], accs[buf], sem_init)
        return idx_cp, init_cp

    descs = [staging(s) for s in range(_NSLOT)]
    descs[0][0].start()
    descs[0][1].start()
    out_cps = [None] * _NSLOT

    def run_slot(s, last):
        chunk = wid + _NW * s
        buf = s % 2
        descs[s][0].wait()
        descs[s][1].wait()
        adds = [
            pltpu.async_copy(fea_hbm.at[idx_v.at[buf, j]], accs[buf],
                             sem_add0, add=True)
            for j in range(_K)
        ]
        if s >= 1:
            out_cps[s - 1].wait()
        if s + 1 < _NSLOT:
            if s + 1 == _NSLOT - 1:
                @pl.when(wid + _NW * (s + 1) < _NFULL)
                def _stage_last():
                    descs[s + 1][0].start()
                    descs[s + 1][1].start()
            else:
                descs[s + 1][0].start()
                descs[s + 1][1].start()
        for cp in adds:
            cp.wait()
        dst = out_hbm.at[pl.ds(chunk * _CB, _CB)]
        if last:
            pltpu.sync_copy(accs[buf], dst)
        else:
            out_cps[s] = pltpu.make_async_copy(accs[buf], dst, sem_add1)
            out_cps[s].start()

    for s in range(_NSLOT - 1):
        run_slot(s, last=False)

    v_last = wid + _NW * (_NSLOT - 1) < _NFULL

    @pl.when(v_last)
    def _last_slot():
        run_slot(_NSLOT - 1, last=True)

    @pl.when(jnp.logical_not(v_last))
    def _drain():
        out_cps[_NSLOT - 2].wait()

    @pl.when(wid == _NFULL % _NW)
    def _tail():
        r0 = _NFULL * _CB
        pltpu.sync_copy(ringT_hbm.at[:, pl.ds(r0, _TAIL)], idx_t)
        pltpu.sync_copy(fea_hbm.at[pl.ds(r0, _TAIL)], acc0.at[pl.ds(0, _TAIL)])
        adds = [
            pltpu.async_copy(fea_hbm.at[idx_t.at[j]],
                             acc0.at[pl.ds(0, _TAIL)], sem_add0, add=True)
            for j in range(_K)
        ]
        for cp in adds:
            cp.wait()
        pltpu.sync_copy(acc0.at[pl.ds(0, _TAIL)], out_hbm.at[pl.ds(r0, _TAIL)])


@functools.cache
def _gather_sum():
    return pl.kernel(
        _gather_sum_body,
        mesh=plsc.VectorSubcoreMesh(core_axis_name="c", subcore_axis_name="s"),
        compiler_params=pltpu.CompilerParams(use_tc_tiling_on_sc=False),
        out_type=jax.ShapeDtypeStruct((_M * _N, _C), jnp.float32),
        scratch_types=[
            pltpu.VMEM((2, _K, _CB), jnp.int32),
            pltpu.VMEM((_CB, _C), jnp.float32),
            pltpu.VMEM((_CB, _C), jnp.float32),
            pltpu.VMEM((_K, _TAIL), jnp.int32),
            pltpu.SemaphoreType.DMA,
            pltpu.SemaphoreType.DMA,
            pltpu.SemaphoreType.DMA,
            pltpu.SemaphoreType.DMA,
        ],
    )


_NMOM = 2


def _post_body(sm_ref, sc_ref, wt_ref, b_ref, g_ref, beta_ref, o_ref,
               ssum_ref, scov_ref, wft_ref, bf_ref):
    i = pl.program_id(0)

    @pl.when(i == 0)
    def _init():
        ssum_ref[...] = jnp.zeros_like(ssum_ref)
        scov_ref[...] = jnp.zeros_like(scov_ref)

    @pl.when(i < _NMOM)
    def _acc():
        s = sm_ref[...]
        ssum_ref[...] += jnp.sum(s, axis=0, keepdims=True)
        scov_ref[...] += lax.dot_general(
            s, s, (((0,), (0,)), ((), ())), preferred_element_type=jnp.float32)

    @pl.when(i == _NMOM - 1)
    def _fold():
        cnt = float(_M * _N)
        wt = wt_ref[...]
        mean_s = ssum_ref[...] / cnt
        outer = lax.dot_general(mean_s, mean_s, (((0,), (0,)), ((), ())),
                                preferred_element_type=jnp.float32)
        cov = scov_ref[...] / cnt - outer
        mu = lax.dot_general(mean_s, wt, (((1,), (0,)), ((), ())),
                             preferred_element_type=jnp.float32) + b_ref[...]
        gt = lax.dot_general(cov, wt, (((1,), (0,)), ((), ())),
                             preferred_element_type=jnp.float32)
        ones = jnp.ones((1, _C), jnp.float32)
        var = lax.dot_general(ones, gt * wt, (((1,), (0,)), ((), ())),
                              preferred_element_type=jnp.float32)
        scale = g_ref[...] * lax.rsqrt(var + 1e-5)
        wft_ref[...] = wt * scale
        bf_ref[...] = (b_ref[...] - mu) * scale + beta_ref[...]

    @pl.when(i >= _NMOM)
    def _conv():
        s = sc_ref[0]
        yT = lax.dot_general(
            wft_ref[...], s, (((0,), (1,)), ((), ())),
            preferred_element_type=jnp.float32)
        ones_n = jnp.ones((1, _N), jnp.float32)
        bias = lax.dot_general(bf_ref[...], ones_n, (((0,), (0,)), ((), ())),
                               preferred_element_type=jnp.float32)
        o_ref[0] = jnp.maximum(yT + bias, 0.0)


def _post(summed, conv_w, conv_b, bn_gamma, bn_beta):
    blk = (_M * _N) // _NMOM
    cc = pl.BlockSpec((_C, _C), lambda i: (0, 0))
    r1 = pl.BlockSpec((1, _C), lambda i: (0, 0))
    return pl.pallas_call(
        _post_body,
        grid=(_NMOM + _M,),
        in_specs=[
            pl.BlockSpec((blk, _C), lambda i: (jnp.minimum(i, _NMOM - 1), 0)),
            pl.BlockSpec((1, _N, _C),
                         lambda i: (jnp.maximum(i - _NMOM, 0), 0, 0)),
            cc, r1, r1, r1,
        ],
        out_specs=pl.BlockSpec((1, _C, _N),
                               lambda i: (jnp.maximum(i - _NMOM, 0), 0, 0)),
        out_shape=jax.ShapeDtypeStruct((_M, _C, _N), jnp.float32),
        scratch_shapes=[
            pltpu.VMEM((1, _C), jnp.float32),
            pltpu.VMEM((_C, _C), jnp.float32),
            pltpu.VMEM((_C, _C), jnp.float32),
            pltpu.VMEM((1, _C), jnp.float32),
        ],
    )(summed, summed.reshape(_M, _N, _C), conv_w.T,
      conv_b[None, :], bn_gamma[None, :], bn_beta[None, :])


def kernel(fea, ring_n, pool_idx, pos_embed, conv_w, conv_b, bn_gamma, bn_beta):
    fea_t = jnp.transpose(fea, (0, 2, 1)).reshape(_M * _N, _C)
    ring_t = (ring_n.astype(jnp.int32) + (jnp.arange(_M, dtype=jnp.int32) * _N)[:, None, None])
    ring_t = jnp.transpose(ring_t, (2, 0, 1)).reshape(_K, _M * _N)

    summed = _gather_sum()(fea_t, ring_t)
    return _post(summed, conv_w, conv_b, bn_gamma, bn_beta)
